# baseline (speedup 1.0000x reference)
import functools
import jax
import jax.numpy as jnp
from jax.experimental import pallas as pl


def _sqdist(src, dst):
    d = -2.0 * jnp.matmul(src, jnp.transpose(dst, (0, 2, 1)))
    d = d + jnp.sum(src ** 2, axis=-1)[:, :, None]
    d = d + jnp.sum(dst ** 2, axis=-1)[:, None, :]
    return d


def _index_points(points, idx):
    B = points.shape[0]
    batch_idx = jnp.arange(B).reshape((B,) + (1,) * (idx.ndim - 1))
    return points[batch_idx, idx]


def _query_ball(radius, nsample, xyz, new_xyz):
    B, N, _ = xyz.shape
    S = new_xyz.shape[1]
    sqrdists = _sqdist(new_xyz, xyz)
    group_idx = jnp.broadcast_to(
        jnp.arange(N, dtype=jnp.int32)[None, None, :], (B, S, N))
    group_idx = jnp.where(sqrdists > radius ** 2, N, group_idx)
    group_idx = jnp.sort(group_idx, axis=-1)[:, :, :nsample]
    group_first = jnp.broadcast_to(group_idx[:, :, :1], (B, S, nsample))
    group_idx = jnp.where(group_idx == N, group_first, group_idx)
    return group_idx


def _bn(x, gamma, beta, axes):
    mean = jnp.mean(x, axis=axes, keepdims=True)
    var = jnp.var(x, axis=axes, keepdims=True)
    xh = (x - mean) / jnp.sqrt(var + 1e-5)
    shape = [1] * x.ndim
    shape[1] = x.shape[1]
    return xh * gamma.reshape(shape) + beta.reshape(shape)


def _fps_body(x_ref, o_ref, *, npoint, n):
    x = x_ref[0]
    iota_np = jax.lax.broadcasted_iota(jnp.int32, (8, npoint), 1)
    iota_n = jax.lax.broadcasted_iota(jnp.int32, (1, n), 1)

    def body(i, carry):
        dist, far, nx = carry
        c = jnp.sum(jnp.where(iota_n == far, x, 0.0), axis=1,
                    keepdims=True)
        nx = jnp.where(iota_np == i, c, nx)
        d = jnp.sum((x - c) ** 2, axis=0, keepdims=True)
        dist = jnp.minimum(dist, d)
        mx = jnp.max(dist)
        far = jnp.min(jnp.where(dist == mx, iota_n, n)).astype(jnp.int32)
        return dist, far, nx

    dist0 = jnp.full((1, n), 1e10, dtype=jnp.float32)
    nx0 = jnp.zeros((8, npoint), dtype=jnp.float32)
    _, _, nx = jax.lax.fori_loop(0, npoint, body,
                                 (dist0, jnp.int32(0), nx0))
    o_ref[0] = nx


def _fps_new_xyz(xyz, npoint):
    B, C, N = xyz.shape
    xp = jnp.concatenate(
        [xyz, jnp.zeros((B, 8 - C, N), xyz.dtype)], axis=1)
    out = pl.pallas_call(
        functools.partial(_fps_body, npoint=npoint, n=N),
        grid=(B,),
        in_specs=[pl.BlockSpec((1, 8, N), lambda b: (b, 0, 0))],
        out_specs=pl.BlockSpec((1, 8, npoint), lambda b: (b, 0, 0)),
        out_shape=jax.ShapeDtypeStruct((B, 8, npoint), jnp.float32),
    )(xp)
    return out[:, :C, :]


def _stage_stats_body(x_ref, w_ref, b_ref, sc_ref, sh_ref,
                      z_ref, s_ref, q_ref, *, act):
    x = x_ref[...] * sc_ref[...] + sh_ref[...]
    if act == 'relu':
        x = jnp.maximum(x, 0.0)
    elif act == 'leaky':
        x = jnp.where(x >= 0, x, 0.01 * x)
    z = jnp.dot(x, w_ref[...], preferred_element_type=jnp.float32)
    z = z + b_ref[...]
    z_ref[...] = z

    @pl.when(pl.program_id(0) == 0)
    def _():
        s_ref[...] = jnp.zeros_like(s_ref)
        q_ref[...] = jnp.zeros_like(q_ref)

    s_ref[...] += jnp.sum(z, axis=0, keepdims=True)
    q_ref[...] += jnp.sum(z * z, axis=0, keepdims=True)


def _stage_plain_body(x_ref, w_ref, b_ref, sc_ref, sh_ref, z_ref, *, act):
    x = x_ref[...] * sc_ref[...] + sh_ref[...]
    if act == 'relu':
        x = jnp.maximum(x, 0.0)
    elif act == 'leaky':
        x = jnp.where(x >= 0, x, 0.01 * x)
    z = jnp.dot(x, w_ref[...], preferred_element_type=jnp.float32)
    z_ref[...] = z + b_ref[...]


def _stage(x, Wt, b, scale, shift, act, want_stats, tile=512):
    R, Ci = x.shape
    Co = Wt.shape[1]
    grid = (R // tile,)
    in_specs = [
        pl.BlockSpec((tile, Ci), lambda i: (i, 0)),
        pl.BlockSpec((Ci, Co), lambda i: (0, 0)),
        pl.BlockSpec((1, Co), lambda i: (0, 0)),
        pl.BlockSpec((1, Ci), lambda i: (0, 0)),
        pl.BlockSpec((1, Ci), lambda i: (0, 0)),
    ]
    if want_stats:
        out = pl.pallas_call(
            functools.partial(_stage_stats_body, act=act),
            grid=grid,
            in_specs=in_specs,
            out_specs=[
                pl.BlockSpec((tile, Co), lambda i: (i, 0)),
                pl.BlockSpec((1, Co), lambda i: (0, 0)),
                pl.BlockSpec((1, Co), lambda i: (0, 0)),
            ],
            out_shape=[
                jax.ShapeDtypeStruct((R, Co), jnp.float32),
                jax.ShapeDtypeStruct((1, Co), jnp.float32),
                jax.ShapeDtypeStruct((1, Co), jnp.float32),
            ],
        )(x, Wt, b, scale, shift)
        return out
    z = pl.pallas_call(
        functools.partial(_stage_plain_body, act=act),
        grid=grid,
        in_specs=in_specs,
        out_specs=pl.BlockSpec((tile, Co), lambda i: (i, 0)),
        out_shape=jax.ShapeDtypeStruct((R, Co), jnp.float32),
    )(x, Wt, b, scale, shift)
    return z


def _bn_scale_shift(s, q, g, be, count):
    m = s[0] / count
    v = q[0] / count - m * m
    sc = g / jnp.sqrt(v + 1e-5)
    sh = be - m * sc
    return sc[None, :], sh[None, :]


def _set_abstraction(xyz, points, npoint, radius, nsample, layers):
    new_xyz_c = _fps_new_xyz(xyz, npoint)
    xyz_t = jnp.transpose(xyz, (0, 2, 1))
    points_t = jnp.transpose(points, (0, 2, 1))
    new_xyz = jnp.transpose(new_xyz_c, (0, 2, 1))
    idx = _query_ball(radius, nsample, xyz_t, new_xyz)
    grouped_xyz = _index_points(xyz_t, idx) - new_xyz[:, :, None, :]
    grouped_points = _index_points(points_t, idx)
    new_points = jnp.concatenate([grouped_xyz, grouped_points], axis=-1)
    new_points = jnp.transpose(new_points, (0, 3, 2, 1))
    for (W, b, g, be) in layers:
        new_points = jnp.einsum('oc,bcks->boks', W, new_points) \
            + b[None, :, None, None]
        new_points = jax.nn.relu(_bn(new_points, g, be, (0, 2, 3)))
    new_points = jnp.max(new_points, axis=2)
    return new_xyz_c, new_points


def _interp3(xyz1, xyz2, points2):
    xyz1_t = jnp.transpose(xyz1, (0, 2, 1))
    xyz2_t = jnp.transpose(xyz2, (0, 2, 1))
    points2_t = jnp.transpose(points2, (0, 2, 1))
    dists = _sqdist(xyz1_t, xyz2_t)
    idx3 = jnp.argsort(dists, axis=-1)[:, :, :3]
    d3 = jnp.take_along_axis(dists, idx3, axis=-1)
    dist_recip = 1.0 / (d3 + 1e-8)
    norm = jnp.sum(dist_recip, axis=2, keepdims=True)
    weight = dist_recip / norm
    return jnp.sum(_index_points(points2_t, idx3) * weight[..., None], axis=2)


def kernel(xyz, params):
    B, _, N = xyz.shape
    l0_points = xyz
    l0_xyz = xyz[:, :3, :]

    l1_xyz, l1_points = _set_abstraction(
        l0_xyz, l0_points, 128, 0.3, 32, params['sa1'])
    l4_xyz, l4_points = _set_abstraction(
        l1_xyz, l1_points, 64, 0.6, 32, params['sa2'])

    interp2 = _interp3(l1_xyz, l4_xyz, l4_points)
    new_points = jnp.concatenate(
        [jnp.transpose(l1_points, (0, 2, 1)), interp2], axis=-1)
    new_points = jnp.transpose(new_points, (0, 2, 1))
    for (W, b, g, be) in params['fp2']:
        new_points = jnp.einsum('oc,bcn->bon', W, new_points) \
            + b[None, :, None]
        new_points = jax.nn.relu(_bn(new_points, g, be, (0, 2)))
    l1_points = new_points

    interp1 = _interp3(l0_xyz, l1_xyz, l1_points)
    R = B * N
    x = interp1.reshape(R, interp1.shape[-1])

    (W1, b1, g1, be1), (W2, b2, g2, be2), (W3, b3, g3, be3) = params['fp1']
    ones = jnp.ones((1, x.shape[1]), jnp.float32)
    zeros = jnp.zeros((1, x.shape[1]), jnp.float32)

    z1, s1, q1 = _stage(x, W1.T, b1[None, :], ones, zeros, 'none', True)
    sc, sh = _bn_scale_shift(s1, q1, g1, be1, R)
    z2, s2, q2 = _stage(z1, W2.T, b2[None, :], sc, sh, 'relu', True)
    sc, sh = _bn_scale_shift(s2, q2, g2, be2, R)
    z3, s3, q3 = _stage(z2, W3.T, b3[None, :], sc, sh, 'relu', True)
    sc, sh = _bn_scale_shift(s3, q3, g3, be3, R)
    z4, s4, q4 = _stage(z3, params['conv1_w'].T,
                        params['conv1_b'][None, :], sc, sh, 'relu', True)
    sc, sh = _bn_scale_shift(s4, q4, params['bn1_g'], params['bn1_b'], R)
    w2p = jnp.zeros((128, 8), jnp.float32).at[:, 0].set(params['conv2_w'][0])
    b2p = jnp.zeros((1, 8), jnp.float32).at[0, 0].set(params['conv2_b'][0])
    z5 = _stage(z4, w2p, b2p, sc, sh, 'leaky', False)

    x_out = z5[:, :1].reshape(B, N, 1).transpose(0, 2, 1)
    return x_out, l4_points


# speedup vs baseline: 1.0099x; 1.0099x over previous
import functools
import jax
import jax.numpy as jnp
from jax.experimental import pallas as pl


def _sqdist(src, dst):
    d = -2.0 * jnp.matmul(src, jnp.transpose(dst, (0, 2, 1)))
    d = d + jnp.sum(src ** 2, axis=-1)[:, :, None]
    d = d + jnp.sum(dst ** 2, axis=-1)[:, None, :]
    return d


def _index_points(points, idx):
    B = points.shape[0]
    batch_idx = jnp.arange(B).reshape((B,) + (1,) * (idx.ndim - 1))
    return points[batch_idx, idx]


def _vpu_cross(a, b):
    p0 = a[:, 0:1] * b[0:1, :]
    p1 = a[:, 1:2] * b[1:2, :]
    p2 = a[:, 2:3] * b[2:3, :]
    return (p0 + p1) + p2


def _vpu_rownorm(a):
    return (a[:, 0:1] ** 2 + a[:, 1:2] ** 2) + a[:, 2:3] ** 2


def _vpu_colnorm(b):
    return (b[0:1, :] ** 2 + b[1:2, :] ** 2) + b[2:3, :] ** 2


def _ball_body(q_ref, x_ref, idx_ref, *, radius, nsample, n):
    q = q_ref[0]
    x = x_ref[0]
    d = -2.0 * _vpu_cross(q, x)
    d = d + _vpu_rownorm(q)
    d = d + _vpu_colnorm(x)
    iota = jax.lax.broadcasted_iota(jnp.int32, (1, n), 1)
    mask = d <= radius ** 2
    cnt = jnp.sum(mask.astype(jnp.int32), axis=-1, keepdims=True)
    idx0 = jnp.min(jnp.where(mask, iota, n), axis=-1, keepdims=True)
    outs = []
    for k in range(nsample):
        ik = jnp.min(jnp.where(mask, iota, n), axis=-1, keepdims=True)
        outs.append(jnp.where(cnt > k, ik, idx0))
        mask = jnp.logical_and(mask, iota != ik)
    idx_ref[0] = jnp.concatenate(outs, axis=-1).astype(jnp.int32)


def _query_ball_pallas(radius, nsample, xyz_p, new_xyz_tp):
    B, _, N = xyz_p.shape
    S = new_xyz_tp.shape[1]
    return pl.pallas_call(
        functools.partial(_ball_body, radius=radius, nsample=nsample, n=N),
        grid=(B,),
        in_specs=[
            pl.BlockSpec((1, S, 8), lambda b: (b, 0, 0)),
            pl.BlockSpec((1, 8, N), lambda b: (b, 0, 0)),
        ],
        out_specs=pl.BlockSpec((1, S, nsample), lambda b: (b, 0, 0)),
        out_shape=jax.ShapeDtypeStruct((B, S, nsample), jnp.int32),
    )(new_xyz_tp, xyz_p)


def _bn(x, gamma, beta, axes):
    mean = jnp.mean(x, axis=axes, keepdims=True)
    var = jnp.var(x, axis=axes, keepdims=True)
    xh = (x - mean) / jnp.sqrt(var + 1e-5)
    shape = [1] * x.ndim
    shape[1] = x.shape[1]
    return xh * gamma.reshape(shape) + beta.reshape(shape)


def _fps_body(x_ref, o_ref, *, npoint, n):
    x = x_ref[0]
    iota_np = jax.lax.broadcasted_iota(jnp.int32, (8, npoint), 1)
    iota_n = jax.lax.broadcasted_iota(jnp.int32, (1, n), 1)

    def body(i, carry):
        dist, far, nx = carry
        c = jnp.sum(jnp.where(iota_n == far, x, 0.0), axis=1,
                    keepdims=True)
        nx = jnp.where(iota_np == i, c, nx)
        d = jnp.sum((x - c) ** 2, axis=0, keepdims=True)
        dist = jnp.minimum(dist, d)
        mx = jnp.max(dist)
        far = jnp.min(jnp.where(dist == mx, iota_n, n)).astype(jnp.int32)
        return dist, far, nx

    dist0 = jnp.full((1, n), 1e10, dtype=jnp.float32)
    nx0 = jnp.zeros((8, npoint), dtype=jnp.float32)
    _, _, nx = jax.lax.fori_loop(0, npoint, body,
                                 (dist0, jnp.int32(0), nx0))
    o_ref[0] = nx


def _fps_new_xyz(xyz_p, npoint):
    B, _, N = xyz_p.shape
    return pl.pallas_call(
        functools.partial(_fps_body, npoint=npoint, n=N),
        grid=(B,),
        in_specs=[pl.BlockSpec((1, 8, N), lambda b: (b, 0, 0))],
        out_specs=pl.BlockSpec((1, 8, npoint), lambda b: (b, 0, 0)),
        out_shape=jax.ShapeDtypeStruct((B, 8, npoint), jnp.float32),
    )(xyz_p)


def _interp_body(x1_ref, x2_ref, p2_ref, o_ref, *, s):
    x1 = x1_ref[0]
    x2 = x2_ref[0]
    p2 = p2_ref[0]
    d = -2.0 * _vpu_cross(x1, x2)
    d = d + _vpu_rownorm(x1)
    d = d + _vpu_colnorm(x2)
    iota = jax.lax.broadcasted_iota(jnp.int32, (1, s), 1)
    acc = jnp.zeros((x1.shape[0], p2.shape[1]), jnp.float32)
    wsum = jnp.zeros((x1.shape[0], 1), jnp.float32)
    for _ in range(3):
        dmin = jnp.min(d, axis=-1, keepdims=True)
        ik = jnp.min(jnp.where(d == dmin, iota, s), axis=-1, keepdims=True)
        r = 1.0 / (dmin + 1e-8)
        onehot = (iota == ik).astype(jnp.float32)
        acc = acc + r * jnp.dot(onehot, p2,
                                preferred_element_type=jnp.float32)
        wsum = wsum + r
        d = jnp.where(iota == ik, jnp.float32(3.0e38), d)
    o_ref[0] = acc / wsum


def _interp_pallas(xyz1_tp, xyz2_p, points2_t, tile=512):
    B, N, _ = xyz1_tp.shape
    S = xyz2_p.shape[2]
    C = points2_t.shape[2]
    return pl.pallas_call(
        functools.partial(_interp_body, s=S),
        grid=(B, N // tile),
        in_specs=[
            pl.BlockSpec((1, tile, 8), lambda b, t: (b, t, 0)),
            pl.BlockSpec((1, 8, S), lambda b, t: (b, 0, 0)),
            pl.BlockSpec((1, S, C), lambda b, t: (b, 0, 0)),
        ],
        out_specs=pl.BlockSpec((1, tile, C), lambda b, t: (b, t, 0)),
        out_shape=jax.ShapeDtypeStruct((B, N, C), jnp.float32),
    )(xyz1_tp, xyz2_p, points2_t)


def _stage_stats_body(x_ref, w_ref, b_ref, sc_ref, sh_ref,
                      z_ref, s_ref, q_ref, *, act):
    x = x_ref[...] * sc_ref[...] + sh_ref[...]
    if act == 'relu':
        x = jnp.maximum(x, 0.0)
    elif act == 'leaky':
        x = jnp.where(x >= 0, x, 0.01 * x)
    z = jnp.dot(x, w_ref[...], preferred_element_type=jnp.float32)
    z = z + b_ref[...]
    z_ref[...] = z

    @pl.when(pl.program_id(0) == 0)
    def _():
        s_ref[...] = jnp.zeros_like(s_ref)
        q_ref[...] = jnp.zeros_like(q_ref)

    s_ref[...] += jnp.sum(z, axis=0, keepdims=True)
    q_ref[...] += jnp.sum(z * z, axis=0, keepdims=True)


def _stage_plain_body(x_ref, w_ref, b_ref, sc_ref, sh_ref, z_ref, *, act):
    x = x_ref[...] * sc_ref[...] + sh_ref[...]
    if act == 'relu':
        x = jnp.maximum(x, 0.0)
    elif act == 'leaky':
        x = jnp.where(x >= 0, x, 0.01 * x)
    z = jnp.dot(x, w_ref[...], preferred_element_type=jnp.float32)
    z_ref[...] = z + b_ref[...]


def _stage(x, Wt, b, scale, shift, act, want_stats, tile=512):
    R, Ci = x.shape
    Co = Wt.shape[1]
    grid = (R // tile,)
    in_specs = [
        pl.BlockSpec((tile, Ci), lambda i: (i, 0)),
        pl.BlockSpec((Ci, Co), lambda i: (0, 0)),
        pl.BlockSpec((1, Co), lambda i: (0, 0)),
        pl.BlockSpec((1, Ci), lambda i: (0, 0)),
        pl.BlockSpec((1, Ci), lambda i: (0, 0)),
    ]
    if want_stats:
        out = pl.pallas_call(
            functools.partial(_stage_stats_body, act=act),
            grid=grid,
            in_specs=in_specs,
            out_specs=[
                pl.BlockSpec((tile, Co), lambda i: (i, 0)),
                pl.BlockSpec((1, Co), lambda i: (0, 0)),
                pl.BlockSpec((1, Co), lambda i: (0, 0)),
            ],
            out_shape=[
                jax.ShapeDtypeStruct((R, Co), jnp.float32),
                jax.ShapeDtypeStruct((1, Co), jnp.float32),
                jax.ShapeDtypeStruct((1, Co), jnp.float32),
            ],
        )(x, Wt, b, scale, shift)
        return out
    z = pl.pallas_call(
        functools.partial(_stage_plain_body, act=act),
        grid=grid,
        in_specs=in_specs,
        out_specs=pl.BlockSpec((tile, Co), lambda i: (i, 0)),
        out_shape=jax.ShapeDtypeStruct((R, Co), jnp.float32),
    )(x, Wt, b, scale, shift)
    return z


def _bn_scale_shift(s, q, g, be, count):
    m = s[0] / count
    v = q[0] / count - m * m
    sc = g / jnp.sqrt(v + 1e-5)
    sh = be - m * sc
    return sc[None, :], sh[None, :]


def _set_abstraction(xyz_p, points, npoint, radius, nsample, layers):
    new_xyz_p = _fps_new_xyz(xyz_p, npoint)
    xyz_t = jnp.transpose(xyz_p[:, :3, :], (0, 2, 1))
    points_t = jnp.transpose(points, (0, 2, 1))
    new_xyz_tp = jnp.transpose(new_xyz_p, (0, 2, 1))
    new_xyz = new_xyz_tp[:, :, :3]
    sqrdists = _sqdist(new_xyz, xyz_t)
    N = xyz_t.shape[1]
    group_idx = jnp.broadcast_to(
        jnp.arange(N, dtype=jnp.int32)[None, None, :],
        (xyz_t.shape[0], npoint, N))
    group_idx = jnp.where(sqrdists > radius ** 2, N, group_idx)
    group_idx = jnp.sort(group_idx, axis=-1)[:, :, :nsample]
    group_first = jnp.broadcast_to(
        group_idx[:, :, :1], group_idx.shape)
    idx = jnp.where(group_idx == N, group_first, group_idx)
    grouped_xyz = _index_points(xyz_t, idx) - new_xyz[:, :, None, :]
    grouped_points = _index_points(points_t, idx)
    new_points = jnp.concatenate([grouped_xyz, grouped_points], axis=-1)
    new_points = jnp.transpose(new_points, (0, 3, 2, 1))
    for (W, b, g, be) in layers:
        new_points = jnp.einsum('oc,bcks->boks', W, new_points) \
            + b[None, :, None, None]
        new_points = jax.nn.relu(_bn(new_points, g, be, (0, 2, 3)))
    new_points = jnp.max(new_points, axis=2)
    return new_xyz_p, new_points


def _interp3(xyz1, xyz2, points2):
    xyz1_t = jnp.transpose(xyz1, (0, 2, 1))
    xyz2_t = jnp.transpose(xyz2, (0, 2, 1))
    points2_t = jnp.transpose(points2, (0, 2, 1))
    dists = _sqdist(xyz1_t, xyz2_t)
    idx3 = jnp.argsort(dists, axis=-1)[:, :, :3]
    d3 = jnp.take_along_axis(dists, idx3, axis=-1)
    dist_recip = 1.0 / (d3 + 1e-8)
    norm = jnp.sum(dist_recip, axis=2, keepdims=True)
    weight = dist_recip / norm
    return jnp.sum(_index_points(points2_t, idx3) * weight[..., None], axis=2)


def kernel(xyz, params):
    B, _, N = xyz.shape
    l0_points = xyz
    l0_xyz = xyz[:, :3, :]
    l0_xyz_p = jnp.concatenate(
        [l0_xyz, jnp.zeros((B, 5, N), jnp.float32)], axis=1)

    l1_xyz_p, l1_points = _set_abstraction(
        l0_xyz_p, l0_points, 128, 0.3, 32, params['sa1'])
    l4_xyz_p, l4_points = _set_abstraction(
        l1_xyz_p, l1_points, 64, 0.6, 32, params['sa2'])

    interp2 = _interp3(l1_xyz_p[:, :3, :], l4_xyz_p[:, :3, :], l4_points)
    new_points = jnp.concatenate(
        [jnp.transpose(l1_points, (0, 2, 1)), interp2], axis=-1)
    new_points = jnp.transpose(new_points, (0, 2, 1))
    for (W, b, g, be) in params['fp2']:
        new_points = jnp.einsum('oc,bcn->bon', W, new_points) \
            + b[None, :, None]
        new_points = jax.nn.relu(_bn(new_points, g, be, (0, 2)))
    l1_points = new_points

    interp1 = _interp3(
        l0_xyz_p[:, :3, :], l1_xyz_p[:, :3, :], l1_points)
    R = B * N
    x = interp1.reshape(R, interp1.shape[-1])

    (W1, b1, g1, be1), (W2, b2, g2, be2), (W3, b3, g3, be3) = params['fp1']
    ones = jnp.ones((1, x.shape[1]), jnp.float32)
    zeros = jnp.zeros((1, x.shape[1]), jnp.float32)

    z1, s1, q1 = _stage(x, W1.T, b1[None, :], ones, zeros, 'none', True)
    sc, sh = _bn_scale_shift(s1, q1, g1, be1, R)
    z2, s2, q2 = _stage(z1, W2.T, b2[None, :], sc, sh, 'relu', True)
    sc, sh = _bn_scale_shift(s2, q2, g2, be2, R)
    z3, s3, q3 = _stage(z2, W3.T, b3[None, :], sc, sh, 'relu', True)
    sc, sh = _bn_scale_shift(s3, q3, g3, be3, R)
    z4, s4, q4 = _stage(z3, params['conv1_w'].T,
                        params['conv1_b'][None, :], sc, sh, 'relu', True)
    sc, sh = _bn_scale_shift(s4, q4, params['bn1_g'], params['bn1_b'], R)
    w2p = jnp.zeros((128, 8), jnp.float32).at[:, 0].set(params['conv2_w'][0])
    b2p = jnp.zeros((1, 8), jnp.float32).at[0, 0].set(params['conv2_b'][0])
    z5 = _stage(z4, w2p, b2p, sc, sh, 'leaky', False)

    x_out = z5[:, :1].reshape(B, N, 1).transpose(0, 2, 1)
    return x_out, l4_points
